# baseline (speedup 1.0000x reference)
import functools

import jax
import jax.numpy as jnp
from jax import lax
from jax.experimental import pallas as pl
from jax.experimental.pallas import tpu as pltpu
from jax.experimental.pallas import tpu_sc as plsc

_BATCH = 16384
_BASE_DIM = 128
_TYPED_DIM = 32
_MULT = 20.0
_PSI = 2.0

_NC = 2
_NS = 16
_L = 16
_NW = _NC * _NS
_BPW = _BATCH // _NW
_C = 64
_NCHUNK = _BPW // _C
_ET_ROWS = 25000


def _sigmoid(x):
    return 1.0 / (1.0 + jnp.exp(-x))


_ET_STEP = 1000


def _compact_body(src_ref, dst_ref):
    for k in range(4):
        dst_ref[:, k * _TYPED_DIM:(k + 1) * _TYPED_DIM] = (
            src_ref[pl.ds(k * _ET_STEP, _ET_STEP), :])


_compact_et = pl.pallas_call(
    _compact_body,
    grid=(_ET_ROWS // _ET_STEP,),
    in_specs=[pl.BlockSpec((4 * _ET_STEP, _TYPED_DIM), lambda i: (i, 0))],
    out_specs=pl.BlockSpec((_ET_STEP, _BASE_DIM), lambda i: (i, 0)),
    out_shape=jax.ShapeDtypeStruct((_ET_ROWS, _BASE_DIM), jnp.float32),
)


def _score_body(s_hbm, r_hbm, o_hbm, e_hbm, r2_hbm, bx_hbm, et4_hbm, out_hbm,
                s_v, r_v, o_v, s4_v, o4_v,
                es_a, eo_a, rr_a, bx_a, st_a, ot_a,
                es_b, eo_b, rr_b, bx_b, st_b, ot_b,
                out_v, sem_a, sem_b):
    wid = lax.axis_index("s") * _NC + lax.axis_index("c")
    base = wid * _BPW
    pltpu.sync_copy(s_hbm.at[pl.ds(base, _BPW)], s_v)
    pltpu.sync_copy(r_hbm.at[pl.ds(base, _BPW)], r_v)
    pltpu.sync_copy(o_hbm.at[pl.ds(base, _BPW)], o_v)

    def idx_body(i, _):
        sl = pl.ds(i * _L, _L)
        sv = s_v[sl]
        ov = o_v[sl]
        s4_v[sl] = lax.div(sv, 4000) * 1000 + lax.rem(sv, 1000)
        o4_v[sl] = lax.div(ov, 4000) * 1000 + lax.rem(ov, 1000)
        return 0

    lax.fori_loop(0, _BPW // _L, idx_body, 0)

    bufs_a = (es_a, eo_a, rr_a, bx_a, st_a, ot_a)
    bufs_b = (es_b, eo_b, rr_b, bx_b, st_b, ot_b)

    def issue(ci, bufs, sem):
        es, eo, rr, bx, st, ot = bufs
        off = ci * _C
        pltpu.async_copy(e_hbm.at[s_v.at[pl.ds(off, _C)]], es, sem)
        pltpu.async_copy(e_hbm.at[o_v.at[pl.ds(off, _C)]], eo, sem)
        pltpu.async_copy(r2_hbm.at[r_v.at[pl.ds(off, _C)]], rr, sem)
        pltpu.async_copy(bx_hbm.at[r_v.at[pl.ds(off, _C)]], bx, sem)
        pltpu.async_copy(et4_hbm.at[s4_v.at[pl.ds(off, _C)]], st, sem)
        pltpu.async_copy(et4_hbm.at[o4_v.at[pl.ds(off, _C)]], ot, sem)

    def drain(bufs, sem):
        es, eo, rr, bx, st, ot = bufs
        pltpu.make_async_copy(e_hbm.at[s_v.at[pl.ds(0, _C)]], es, sem).wait()
        pltpu.make_async_copy(e_hbm.at[o_v.at[pl.ds(0, _C)]], eo, sem).wait()
        pltpu.make_async_copy(r2_hbm.at[r_v.at[pl.ds(0, _C)]], rr, sem).wait()
        pltpu.make_async_copy(bx_hbm.at[r_v.at[pl.ds(0, _C)]], bx, sem).wait()
        pltpu.make_async_copy(et4_hbm.at[s4_v.at[pl.ds(0, _C)]], st, sem).wait()
        pltpu.make_async_copy(et4_hbm.at[o4_v.at[pl.ds(0, _C)]], ot, sem).wait()

    def compute(ci, bufs):
        es, eo, rr, bx, st, ot = bufs
        off = ci * _C
        for g in range(_C // _L):
            rows = lax.iota(jnp.int32, _L) + (g * _L)
            lanes = lax.iota(jnp.int32, _L)
            s16 = s_v[pl.ds(off + g * _L, _L)]
            o16 = o_v[pl.ds(off + g * _L, _L)]
            srem = lax.shift_left(jnp.bitwise_and(lax.div(s16, 1000), 3), 5)
            orem = lax.shift_left(jnp.bitwise_and(lax.div(o16, 1000), 3), 5)

            def base_dot(d, acc):
                col = jnp.bitwise_and(lanes + d, _BASE_DIM - 1)
                a = plsc.load_gather(es, [rows, col])
                b = plsc.load_gather(rr, [rows, col])
                c = plsc.load_gather(eo, [rows, col])
                return acc + a * b * c

            acc0 = jnp.zeros((_L,), jnp.float32)
            base_acc = lax.fori_loop(0, _BASE_DIM, base_dot, acc0, unroll=8)

            def typed_step(d, carry):
                hmax, pps, pls, phs, tmax, ppo, plo, pho = carry
                col = jnp.bitwise_and(lanes + d, _TYPED_DIM - 1)
                p_s = plsc.load_gather(st, [rows, srem + col])
                p_o = plsc.load_gather(ot, [rows, orem + col])
                lo_h = plsc.load_gather(bx, [rows, col])
                hi_h = plsc.load_gather(bx, [rows, col + _TYPED_DIM])
                lo_t = plsc.load_gather(bx, [rows, col + 2 * _TYPED_DIM])
                hi_t = plsc.load_gather(bx, [rows, col + 3 * _TYPED_DIM])
                hmax = jnp.maximum(
                    hmax, jnp.maximum(jnp.maximum(lo_h - p_s, 0.0), p_s - hi_h))
                tmax = jnp.maximum(
                    tmax, jnp.maximum(jnp.maximum(lo_t - p_o, 0.0), p_o - hi_t))
                pps = pps + p_s * p_s
                pls = pls + p_s * lo_h
                phs = phs + p_s * hi_h
                ppo = ppo + p_o * p_o
                plo = plo + p_o * lo_t
                pho = pho + p_o * hi_t
                return hmax, pps, pls, phs, tmax, ppo, plo, pho

            z = jnp.zeros((_L,), jnp.float32)
            carry0 = (z, z, z, z, z, z, z, z)
            hmax, pps, pls, phs, tmax, ppo, plo, pho = lax.fori_loop(
                0, _TYPED_DIM, typed_step, carry0, unroll=4)

            dist_h = jnp.where(hmax > 0.0, jnp.maximum(pls, phs), pps)
            dist_t = jnp.where(tmax > 0.0, jnp.maximum(plo, pho), ppo)
            res = (_MULT * _sigmoid(_PSI * base_acc)
                   * _sigmoid(-_PSI * dist_h) * _sigmoid(-_PSI * dist_t))
            out_v[pl.ds(off + g * _L, _L)] = res

    issue(0, bufs_a, sem_a)

    def pair_body(i, _):
        c0 = 2 * i
        issue(c0 + 1, bufs_b, sem_b)
        drain(bufs_a, sem_a)
        compute(c0, bufs_a)

        @pl.when(c0 + 2 < _NCHUNK)
        def _():
            issue(c0 + 2, bufs_a, sem_a)

        drain(bufs_b, sem_b)
        compute(c0 + 1, bufs_b)
        return 0

    lax.fori_loop(0, _NCHUNK // 2, pair_body, 0)
    pltpu.sync_copy(out_v, out_hbm.at[pl.ds(base, _BPW)])


_mesh = plsc.VectorSubcoreMesh(
    core_axis_name="c", subcore_axis_name="s",
    num_cores=_NC, num_subcores=_NS)

_chunk_bufs = [
    pltpu.VMEM((_C, _BASE_DIM), jnp.float32),
    pltpu.VMEM((_C, _BASE_DIM), jnp.float32),
    pltpu.VMEM((_C, _BASE_DIM), jnp.float32),
    pltpu.VMEM((_C, _BASE_DIM), jnp.float32),
    pltpu.VMEM((_C, _BASE_DIM), jnp.float32),
    pltpu.VMEM((_C, _BASE_DIM), jnp.float32),
]

_score = functools.partial(
    pl.kernel,
    out_type=jax.ShapeDtypeStruct((_BATCH,), jnp.float32),
    mesh=_mesh,
    scratch_types=[
        pltpu.VMEM((_BPW,), jnp.int32),
        pltpu.VMEM((_BPW,), jnp.int32),
        pltpu.VMEM((_BPW,), jnp.int32),
        pltpu.VMEM((_BPW,), jnp.int32),
        pltpu.VMEM((_BPW,), jnp.int32),
        *_chunk_bufs,
        *_chunk_bufs,
        pltpu.VMEM((_BPW,), jnp.float32),
        pltpu.SemaphoreType.DMA,
        pltpu.SemaphoreType.DMA,
    ],
    compiler_params=pltpu.CompilerParams(needs_layout_passes=False),
)(_score_body)


@jax.jit
def kernel(s, r, o, E, R, E_t, R_ht_low, R_ht_high, R_tt_low, R_tt_high):
    boxes = jnp.concatenate([R_ht_low, R_ht_high, R_tt_low, R_tt_high], axis=1)
    et4 = _compact_et(E_t)
    return _score(s.astype(jnp.int32), r.astype(jnp.int32), o.astype(jnp.int32),
                  E, R, boxes, et4)


# speedup vs baseline: 1.2872x; 1.2872x over previous
import functools

import jax
import jax.numpy as jnp
from jax import lax
from jax.experimental import pallas as pl
from jax.experimental.pallas import tpu as pltpu
from jax.experimental.pallas import tpu_sc as plsc

_BATCH = 16384
_BASE_DIM = 128
_TYPED_DIM = 32
_MULT = 20.0
_PSI = 2.0

_NC = 2
_NS = 16
_L = 16
_NW = _NC * _NS
_BPW = _BATCH // _NW
_C = 64
_NCHUNK = _BPW // _C
_ET_ROWS = 25000

_mesh = plsc.VectorSubcoreMesh(
    core_axis_name="c", subcore_axis_name="s",
    num_cores=_NC, num_subcores=_NS)


def _sigmoid(x):
    return 1.0 / (1.0 + jnp.exp(-x))


def _worker_base():
    return (lax.axis_index("s") * _NC + lax.axis_index("c")) * _BPW


def _base_body(s_hbm, r_hbm, o_hbm, e_hbm, r2_hbm, out_hbm,
               s_v, r_v, o_v,
               es_a, eo_a, rr_a, es_b, eo_b, rr_b,
               out_v, sem_a, sem_b):
    base = _worker_base()
    pltpu.sync_copy(s_hbm.at[pl.ds(base, _BPW)], s_v)
    pltpu.sync_copy(r_hbm.at[pl.ds(base, _BPW)], r_v)
    pltpu.sync_copy(o_hbm.at[pl.ds(base, _BPW)], o_v)

    def issue(ci, bufs, sem):
        es, eo, rr = bufs
        off = ci * _C
        pltpu.async_copy(e_hbm.at[s_v.at[pl.ds(off, _C)]], es, sem)
        pltpu.async_copy(e_hbm.at[o_v.at[pl.ds(off, _C)]], eo, sem)
        pltpu.async_copy(r2_hbm.at[r_v.at[pl.ds(off, _C)]], rr, sem)

    def drain(bufs, sem):
        es, eo, rr = bufs
        pltpu.make_async_copy(e_hbm.at[s_v.at[pl.ds(0, _C)]], es, sem).wait()
        pltpu.make_async_copy(e_hbm.at[o_v.at[pl.ds(0, _C)]], eo, sem).wait()
        pltpu.make_async_copy(r2_hbm.at[r_v.at[pl.ds(0, _C)]], rr, sem).wait()

    def compute(ci, bufs):
        es, eo, rr = bufs
        off = ci * _C
        for g in range(_C // _L):
            rows = lax.iota(jnp.int32, _L) + (g * _L)
            lanes = lax.iota(jnp.int32, _L)

            def base_dot(d, acc):
                col = jnp.bitwise_and(lanes + d, _BASE_DIM - 1)
                a = plsc.load_gather(es, [rows, col])
                b = plsc.load_gather(rr, [rows, col])
                c = plsc.load_gather(eo, [rows, col])
                return acc + a * b * c

            acc0 = jnp.zeros((_L,), jnp.float32)
            acc = lax.fori_loop(0, _BASE_DIM, base_dot, acc0, unroll=8)
            out_v[pl.ds(off + g * _L, _L)] = _MULT * _sigmoid(_PSI * acc)

    bufs0 = (es_a, eo_a, rr_a)
    bufs1 = (es_b, eo_b, rr_b)
    issue(0, bufs0, sem_a)

    def pair(i, _):
        c0 = 2 * i
        issue(c0 + 1, bufs1, sem_b)
        drain(bufs0, sem_a)
        compute(c0, bufs0)

        @pl.when(c0 + 2 < _NCHUNK)
        def _():
            issue(c0 + 2, bufs0, sem_a)

        drain(bufs1, sem_b)
        compute(c0 + 1, bufs1)
        return 0

    lax.fori_loop(0, _NCHUNK // 2, pair, 0)
    pltpu.sync_copy(out_v, out_hbm.at[pl.ds(base, _BPW)])


_base_score = functools.partial(
    pl.kernel,
    out_type=jax.ShapeDtypeStruct((_BATCH,), jnp.float32),
    mesh=_mesh,
    scratch_types=[
        pltpu.VMEM((_BPW,), jnp.int32),
        pltpu.VMEM((_BPW,), jnp.int32),
        pltpu.VMEM((_BPW,), jnp.int32),
        pltpu.VMEM((_C, _BASE_DIM), jnp.float32),
        pltpu.VMEM((_C, _BASE_DIM), jnp.float32),
        pltpu.VMEM((_C, _BASE_DIM), jnp.float32),
        pltpu.VMEM((_C, _BASE_DIM), jnp.float32),
        pltpu.VMEM((_C, _BASE_DIM), jnp.float32),
        pltpu.VMEM((_C, _BASE_DIM), jnp.float32),
        pltpu.VMEM((_BPW,), jnp.float32),
        pltpu.SemaphoreType.DMA,
        pltpu.SemaphoreType.DMA,
    ],
    compiler_params=pltpu.CompilerParams(needs_layout_passes=False),
)(_base_body)


def _typed_body(s_hbm, r_hbm, o_hbm, bx_hbm, et4_hbm, part_hbm, out_hbm,
                s_v, r_v, o_v, s4_v, o4_v,
                bx_a, st_a, ot_a, bx_b, st_b, ot_b,
                out_v, sem_a, sem_b):
    base = _worker_base()
    pltpu.sync_copy(s_hbm.at[pl.ds(base, _BPW)], s_v)
    pltpu.sync_copy(r_hbm.at[pl.ds(base, _BPW)], r_v)
    pltpu.sync_copy(o_hbm.at[pl.ds(base, _BPW)], o_v)
    pltpu.sync_copy(part_hbm.at[pl.ds(base, _BPW)], out_v)

    def idx_body(i, _):
        sl = pl.ds(i * _L, _L)
        s4_v[sl] = lax.shift_right_logical(s_v[sl], 2)
        o4_v[sl] = lax.shift_right_logical(o_v[sl], 2)
        return 0

    lax.fori_loop(0, _BPW // _L, idx_body, 0)

    def issue(ci, bufs, sem):
        bx, st, ot = bufs
        off = ci * _C
        pltpu.async_copy(bx_hbm.at[r_v.at[pl.ds(off, _C)]], bx, sem)
        pltpu.async_copy(et4_hbm.at[s4_v.at[pl.ds(off, _C)]], st, sem)
        pltpu.async_copy(et4_hbm.at[o4_v.at[pl.ds(off, _C)]], ot, sem)

    def drain(bufs, sem):
        bx, st, ot = bufs
        pltpu.make_async_copy(bx_hbm.at[r_v.at[pl.ds(0, _C)]], bx, sem).wait()
        pltpu.make_async_copy(et4_hbm.at[s4_v.at[pl.ds(0, _C)]], st, sem).wait()
        pltpu.make_async_copy(et4_hbm.at[o4_v.at[pl.ds(0, _C)]], ot, sem).wait()

    def compute(ci, bufs):
        bx, st, ot = bufs
        off = ci * _C
        for g in range(_C // _L):
            rows = lax.iota(jnp.int32, _L) + (g * _L)
            lanes = lax.iota(jnp.int32, _L)
            s16 = s_v[pl.ds(off + g * _L, _L)]
            o16 = o_v[pl.ds(off + g * _L, _L)]
            srem = lax.shift_left(jnp.bitwise_and(s16, 3), 5)
            orem = lax.shift_left(jnp.bitwise_and(o16, 3), 5)

            def typed_step(d, carry):
                hmax, pps, pls, phs, tmax, ppo, plo, pho = carry
                col = jnp.bitwise_and(lanes + d, _TYPED_DIM - 1)
                p_s = plsc.load_gather(st, [rows, srem + col])
                p_o = plsc.load_gather(ot, [rows, orem + col])
                lo_h = plsc.load_gather(bx, [rows, col])
                hi_h = plsc.load_gather(bx, [rows, col + _TYPED_DIM])
                lo_t = plsc.load_gather(bx, [rows, col + 2 * _TYPED_DIM])
                hi_t = plsc.load_gather(bx, [rows, col + 3 * _TYPED_DIM])
                hmax = jnp.maximum(
                    hmax, jnp.maximum(jnp.maximum(lo_h - p_s, 0.0), p_s - hi_h))
                tmax = jnp.maximum(
                    tmax, jnp.maximum(jnp.maximum(lo_t - p_o, 0.0), p_o - hi_t))
                pps = pps + p_s * p_s
                pls = pls + p_s * lo_h
                phs = phs + p_s * hi_h
                ppo = ppo + p_o * p_o
                plo = plo + p_o * lo_t
                pho = pho + p_o * hi_t
                return hmax, pps, pls, phs, tmax, ppo, plo, pho

            z = jnp.zeros((_L,), jnp.float32)
            carry0 = (z, z, z, z, z, z, z, z)
            hmax, pps, pls, phs, tmax, ppo, plo, pho = lax.fori_loop(
                0, _TYPED_DIM, typed_step, carry0, unroll=4)

            dist_h = jnp.where(hmax > 0.0, jnp.maximum(pls, phs), pps)
            dist_t = jnp.where(tmax > 0.0, jnp.maximum(plo, pho), ppo)
            sl = pl.ds(off + g * _L, _L)
            out_v[sl] = (out_v[sl]
                         * _sigmoid(-_PSI * dist_h) * _sigmoid(-_PSI * dist_t))

    bufs0 = (bx_a, st_a, ot_a)
    bufs1 = (bx_b, st_b, ot_b)
    issue(0, bufs0, sem_a)

    def pair(i, _):
        c0 = 2 * i
        issue(c0 + 1, bufs1, sem_b)
        drain(bufs0, sem_a)
        compute(c0, bufs0)

        @pl.when(c0 + 2 < _NCHUNK)
        def _():
            issue(c0 + 2, bufs0, sem_a)

        drain(bufs1, sem_b)
        compute(c0 + 1, bufs1)
        return 0

    lax.fori_loop(0, _NCHUNK // 2, pair, 0)
    pltpu.sync_copy(out_v, out_hbm.at[pl.ds(base, _BPW)])


_typed_score = functools.partial(
    pl.kernel,
    out_type=jax.ShapeDtypeStruct((_BATCH,), jnp.float32),
    mesh=_mesh,
    scratch_types=[
        pltpu.VMEM((_BPW,), jnp.int32),
        pltpu.VMEM((_BPW,), jnp.int32),
        pltpu.VMEM((_BPW,), jnp.int32),
        pltpu.VMEM((_BPW,), jnp.int32),
        pltpu.VMEM((_BPW,), jnp.int32),
        pltpu.VMEM((_C, _BASE_DIM), jnp.float32),
        pltpu.VMEM((_C, _BASE_DIM), jnp.float32),
        pltpu.VMEM((_C, _BASE_DIM), jnp.float32),
        pltpu.VMEM((_C, _BASE_DIM), jnp.float32),
        pltpu.VMEM((_C, _BASE_DIM), jnp.float32),
        pltpu.VMEM((_C, _BASE_DIM), jnp.float32),
        pltpu.VMEM((_BPW,), jnp.float32),
        pltpu.SemaphoreType.DMA,
        pltpu.SemaphoreType.DMA,
    ],
    compiler_params=pltpu.CompilerParams(needs_layout_passes=False),
)(_typed_body)


@jax.jit
def kernel(s, r, o, E, R, E_t, R_ht_low, R_ht_high, R_tt_low, R_tt_high):
    s = s.astype(jnp.int32)
    r = r.astype(jnp.int32)
    o = o.astype(jnp.int32)
    boxes = jnp.concatenate([R_ht_low, R_ht_high, R_tt_low, R_tt_high], axis=1)
    et4 = jnp.reshape(E_t, (_ET_ROWS, _BASE_DIM))
    part = _base_score(s, r, o, E, R)
    return _typed_score(s, r, o, boxes, et4, part)
